# baseline (speedup 1.0000x reference)
import jax
import jax.numpy as jnp
from jax.experimental import pallas as pl
from jax.experimental.pallas import tpu as pltpu


def _split_body(e_ref, c3d_ref, cbn_ref):
    e = e_ref[...]
    c1 = e.astype(jnp.bfloat16)
    r1 = e - c1.astype(jnp.float32)
    c2 = r1.astype(jnp.bfloat16)
    r2 = r1 - c2.astype(jnp.float32)
    c3d_ref[0] = jnp.concatenate(
        [c1[0], c2[0], r2[0].astype(jnp.bfloat16)], axis=1)
    cb = e[0]
    cbn = jnp.sum(cb * cb, axis=1)
    cbn_ref[0] = jnp.broadcast_to(cbn[:, None], cbn_ref.shape[1:])


def _rvq_body(x_ref, c3d_ref, cbn_ref, out_ref, codes_ref):
    Q, K, D3 = c3d_ref.shape
    D = D3 // 3
    Tb = x_ref.shape[2]
    r = x_ref[0]
    qt = jnp.zeros_like(r)
    iota = jax.lax.broadcasted_iota(jnp.int32, (K, Tb), 0)
    for q in range(Q):
        cbn = cbn_ref[q][:, :1]
        scores = jax.lax.dot_general(
            c3d_ref[q][:, :D], r.astype(jnp.bfloat16),
            (((1,), (0,)), ((), ())),
            preferred_element_type=jnp.float32)
        rn = jnp.sum(r * r, axis=0)
        dist = (rn[None, :] - 2.0 * scores) + cbn
        mn = jnp.min(dist, axis=0)
        idx = jnp.min(jnp.where(dist == mn[None, :], iota, K), axis=0)
        oh = (iota == idx[None, :]).astype(jnp.bfloat16)
        y = jax.lax.dot_general(
            c3d_ref[q], oh, (((0,), (0,)), ((), ())),
            preferred_element_type=jnp.float32)
        quant = (y[:D] + y[D:2 * D]) + y[2 * D:]
        r = r - quant
        qt = qt + quant
        codes_ref[0, 0, q, :] = idx
    out_ref[0] = qt


def kernel(embeddings, embed):
    B, D, T = embeddings.shape
    Q, K, _ = embed.shape
    c3d, cbn = pl.pallas_call(
        _split_body,
        grid=(Q,),
        in_specs=[pl.BlockSpec((1, K, D), lambda q: (q, 0, 0))],
        out_specs=(
            pl.BlockSpec((1, K, 3 * D), lambda q: (q, 0, 0)),
            pl.BlockSpec((1, K, 128), lambda q: (q, 0, 0)),
        ),
        out_shape=(
            jax.ShapeDtypeStruct((Q, K, 3 * D), jnp.bfloat16),
            jax.ShapeDtypeStruct((Q, K, 128), jnp.float32),
        ),
    )(embed)
    Tb = 2048
    grid = (B, T // Tb)
    qout, codes4 = pl.pallas_call(
        _rvq_body,
        grid=grid,
        in_specs=[
            pl.BlockSpec((1, D, Tb), lambda b, t: (b, 0, t)),
            pl.BlockSpec((Q, K, 3 * D), lambda b, t: (0, 0, 0)),
            pl.BlockSpec((Q, K, 128), lambda b, t: (0, 0, 0)),
        ],
        out_specs=(
            pl.BlockSpec((1, D, Tb), lambda b, t: (b, 0, t)),
            pl.BlockSpec((1, 1, Q, Tb), lambda b, t: (b, t, 0, 0)),
        ),
        out_shape=(
            jax.ShapeDtypeStruct((B, D, T), jnp.float32),
            jax.ShapeDtypeStruct((B, T // Tb, Q, Tb), jnp.int32),
        ),
        compiler_params=pltpu.CompilerParams(
            dimension_semantics=("parallel", "parallel")),
    )(embeddings, c3d, cbn)
    codes = jnp.transpose(codes4, (2, 0, 1, 3)).reshape(Q, B, T)
    return (qout, codes)


# speedup vs baseline: 1.4187x; 1.4187x over previous
import jax
import jax.numpy as jnp
from jax.experimental import pallas as pl
from jax.experimental.pallas import tpu as pltpu


def _split_body(e_ref, c3d_ref, cbn_ref):
    e = e_ref[...]
    c1 = e.astype(jnp.bfloat16)
    r1 = e - c1.astype(jnp.float32)
    c2 = r1.astype(jnp.bfloat16)
    r2 = r1 - c2.astype(jnp.float32)
    c3d_ref[0] = jnp.concatenate(
        [c1[0], c2[0], r2[0].astype(jnp.bfloat16)], axis=1)
    cb = e[0]
    cbn = jnp.sum(cb * cb, axis=1)
    cbn_ref[0] = jnp.broadcast_to(cbn[:, None], cbn_ref.shape[1:])


def _rvq_body(x_ref, c3d_ref, cbn_ref, out_ref, codes_ref):
    Q, K, D3 = c3d_ref.shape
    D = D3 // 3
    Tb = x_ref.shape[2]
    r = x_ref[0]
    qt = jnp.zeros_like(r)
    iota = jax.lax.broadcasted_iota(jnp.int32, (K, Tb), 0)
    for q in range(Q):
        cbn = cbn_ref[q][:, :1]
        scores = jax.lax.dot_general(
            c3d_ref[q][:, :D], r.astype(jnp.bfloat16),
            (((1,), (0,)), ((), ())),
            preferred_element_type=jnp.float32)
        rn = jnp.sum(r * r, axis=0)
        S = 32
        rn_row = rn[None, :]
        d0 = (rn_row - 2.0 * scores[0:S]) + cbn[0:S]
        run_m = d0
        run_c = jnp.zeros((S, Tb), jnp.int32)
        for c in range(1, K // S):
            sl = slice(c * S, (c + 1) * S)
            d = (rn_row - 2.0 * scores[sl]) + cbn[sl]
            better = d < run_m
            run_m = jnp.where(better, d, run_m)
            run_c = jnp.where(better, c, run_c)
        iota_s = jax.lax.broadcasted_iota(jnp.int32, (S, Tb), 0)
        m, kk = run_m, run_c * S + iota_s
        size = S
        while size > 1:
            h = size // 2
            mA, mB = m[:h], m[h:size]
            kA, kB = kk[:h], kk[h:size]
            takeB = (mB < mA) | ((mB == mA) & (kB < kA))
            m = jnp.where(takeB, mB, mA)
            kk = jnp.where(takeB, kB, kA)
            size = h
        idx = kk[0]
        oh = (iota == idx[None, :]).astype(jnp.bfloat16)
        y = jax.lax.dot_general(
            c3d_ref[q], oh, (((0,), (0,)), ((), ())),
            preferred_element_type=jnp.float32)
        quant = (y[:D] + y[D:2 * D]) + y[2 * D:]
        r = r - quant
        qt = qt + quant
        codes_ref[0, 0, q, :] = idx
    out_ref[0] = qt


def kernel(embeddings, embed):
    B, D, T = embeddings.shape
    Q, K, _ = embed.shape
    c3d, cbn = pl.pallas_call(
        _split_body,
        grid=(Q,),
        in_specs=[pl.BlockSpec((1, K, D), lambda q: (q, 0, 0))],
        out_specs=(
            pl.BlockSpec((1, K, 3 * D), lambda q: (q, 0, 0)),
            pl.BlockSpec((1, K, 128), lambda q: (q, 0, 0)),
        ),
        out_shape=(
            jax.ShapeDtypeStruct((Q, K, 3 * D), jnp.bfloat16),
            jax.ShapeDtypeStruct((Q, K, 128), jnp.float32),
        ),
    )(embed)
    Tb = 1024
    grid = (B, T // Tb)
    qout, codes4 = pl.pallas_call(
        _rvq_body,
        grid=grid,
        in_specs=[
            pl.BlockSpec((1, D, Tb), lambda b, t: (b, 0, t)),
            pl.BlockSpec((Q, K, 3 * D), lambda b, t: (0, 0, 0)),
            pl.BlockSpec((Q, K, 128), lambda b, t: (0, 0, 0)),
        ],
        out_specs=(
            pl.BlockSpec((1, D, Tb), lambda b, t: (b, 0, t)),
            pl.BlockSpec((1, 1, Q, Tb), lambda b, t: (b, t, 0, 0)),
        ),
        out_shape=(
            jax.ShapeDtypeStruct((B, D, T), jnp.float32),
            jax.ShapeDtypeStruct((B, T // Tb, Q, Tb), jnp.int32),
        ),
        compiler_params=pltpu.CompilerParams(
            dimension_semantics=("parallel", "parallel")),
    )(embeddings, c3d, cbn)
    codes = jnp.transpose(codes4, (2, 0, 1, 3)).reshape(Q, B, T)
    return (qout, codes)
